# Initial kernel scaffold; baseline (speedup 1.0000x reference)
import jax
import jax.numpy as jnp
from jax.experimental import pallas as pl


def kernel(x, edge_index, weight, W1, b1, W2, b2, W3, b3, W4, b4, W5, b5, W6, b6, W7, b7, W8, b8, W9, b9):
    raise NotImplementedError("write your pallas kernel here")


# speedup vs baseline: 4.5460x; 4.5460x over previous
import functools

import jax
import jax.numpy as jnp
from jax import lax
from jax.experimental import pallas as pl
from jax.experimental.pallas import tpu as pltpu
from jax.experimental.pallas import tpu_sc as plsc

N = 10000
NPAD = 10240
E = 320000
E2 = E + N
NC, NS = 2, 16
NTILES = NC * NS
K = 128
EP = ((E2 + NTILES * K - 1) // (NTILES * K)) * (NTILES * K)
ROWS_W = NPAD // NS
WCH = 128
F = 128

_MESH = plsc.VectorSubcoreMesh(
    core_axis_name="c", subcore_axis_name="s", num_cores=NC, num_subcores=NS)


def _zero_buf(buf, nrows, width):
    def zb(r, _):
        for i in range(width // 16):
            buf[r, pl.ds(i * 16, 16)] = jnp.zeros((16,), jnp.float32)
        return 0
    lax.fori_loop(0, nrows, zb, 0)


def _make_deg_kernel():
    per_tile = EP // NTILES
    nch = per_tile // K

    def body(meta_hbm, ew_hbm, out_hbm, meta_v, ewc_v, dstc_v, rows_v, wbuf,
             acc):
        c = lax.axis_index("c")
        s = lax.axis_index("s")
        tile = c * NS + s
        _zero_buf(wbuf, WCH, F)

        def zacc(w, _):
            pltpu.sync_copy(wbuf, acc.at[pl.ds(s * ROWS_W + w * WCH, WCH)])
            return 0
        lax.fori_loop(0, ROWS_W // WCH, zacc, 0)
        plsc.subcore_barrier()

        def chunk(g, _):
            mbase = (tile * nch + g) * (2 * K)
            pltpu.sync_copy(meta_hbm.at[pl.ds(mbase, 2 * K)], meta_v)
            pltpu.sync_copy(ew_hbm.at[pl.ds((tile * nch + g) * K, K)], ewc_v)

            def grp(gg, _):
                dstc_v[pl.ds(gg * 16, 16)] = meta_v[pl.ds(K + gg * 16, 16)]
                wv16 = ewc_v[pl.ds(gg * 16, 16)]
                for lane in range(16):
                    wv = jnp.full((16,), wv16[lane], jnp.float32)
                    for i in range(F // 16):
                        rows_v[gg * 16 + lane, pl.ds(i * 16, 16)] = wv
                return 0
            lax.fori_loop(0, K // 16, grp, 0)
            pltpu.sync_copy(rows_v, acc.at[dstc_v], add=True)
            return 0
        lax.fori_loop(0, nch, chunk, 0)
        plsc.subcore_barrier()

        def wout(w, _):
            base = s * ROWS_W + w * WCH
            pltpu.sync_copy(acc.at[pl.ds(base, WCH)], wbuf)
            pltpu.sync_copy(wbuf, out_hbm.at[pl.ds(c * NPAD + base, WCH)])
            return 0
        lax.fori_loop(0, ROWS_W // WCH, wout, 0)

    return pl.kernel(
        body,
        out_type=jax.ShapeDtypeStruct((2 * NPAD, F), jnp.float32),
        mesh=_MESH,
        scratch_types=[
            pltpu.VMEM((2 * K,), jnp.int32),
            pltpu.VMEM((K,), jnp.float32),
            pltpu.VMEM((K,), jnp.int32),
            pltpu.VMEM((K, F), jnp.float32),
            pltpu.VMEM((WCH, F), jnp.float32),
            pltpu.VMEM_SHARED((NPAD, F), jnp.float32),
        ],
    )


def _dinv_tc(deg2):
    def body(d_ref, o_ref):
        dsum = d_ref[0] + d_ref[1]
        o_ref[...] = jnp.where(
            dsum > 0, lax.rsqrt(jnp.maximum(dsum, 1e-12)), 0.0)

    return pl.pallas_call(
        body,
        out_shape=jax.ShapeDtypeStruct((NPAD, 128), jnp.float32),
    )(deg2)


def _make_agg_kernel(split_mode, with_epi):
    if split_mode:
        per_tile = EP // NTILES
        out_rows = 2 * NPAD
        npass = 1
    else:
        per_tile = EP // NS
        out_rows = 4 * NPAD
        npass = 2
    nch = per_tile // K

    def body(hw_hbm, meta_hbm, ew_hbm, dinv_hbm, bias_hbm, out_hbm,
             meta_v, ewc_v, dstc_v, gidx_v, rows_v, wbuf, bias_v,
             acc, sem):
        c = lax.axis_index("c")
        s = lax.axis_index("s")
        tile = (c * NS + s) if split_mode else s

        for p in range(npass):
            fb = c * 2 + p
            if with_epi:
                pltpu.sync_copy(bias_hbm.at[pl.ds(fb * F, F)], bias_v)
            _zero_buf(wbuf, WCH, F)

            def zacc(w, _):
                pltpu.sync_copy(wbuf,
                                acc.at[pl.ds(s * ROWS_W + w * WCH, WCH)])
                return 0
            lax.fori_loop(0, ROWS_W // WCH, zacc, 0)
            plsc.subcore_barrier()
            offs = fb * NPAD if not split_mode else 0

            def chunk(g, _):
                mbase = (tile * nch + g) * (2 * K)
                pltpu.sync_copy(meta_hbm.at[pl.ds(mbase, 2 * K)], meta_v)
                pltpu.sync_copy(
                    ew_hbm.at[pl.ds((tile * nch + g) * K, K)], ewc_v)

                def grp(gg, _):
                    sl16 = pl.ds(gg * 16, 16)
                    gidx_v[sl16] = meta_v[sl16] + offs
                    dstc_v[sl16] = meta_v[pl.ds(K + gg * 16, 16)]
                    return 0
                lax.fori_loop(0, K // 16, grp, 0)
                pltpu.async_copy(hw_hbm.at[gidx_v], rows_v, sem).wait()

                def grp16(gg, _):
                    ew16 = ewc_v[pl.ds(gg * 16, 16)]
                    for lane in range(16):
                        e = gg * 16 + lane
                        nv = jnp.full((16,), ew16[lane], jnp.float32)
                        for i in range(F // 16):
                            sl = pl.ds(i * 16, 16)
                            rows_v[e, sl] = rows_v[e, sl] * nv
                    return 0
                lax.fori_loop(0, K // 16, grp16, 0)
                pltpu.sync_copy(rows_v, acc.at[dstc_v], add=True)
                return 0
            lax.fori_loop(0, nch, chunk, 0)
            plsc.subcore_barrier()

            obase = (c * NPAD if split_mode else fb * NPAD) + s * ROWS_W

            def wout(w, _):
                nbase = s * ROWS_W + w * WCH
                pltpu.sync_copy(acc.at[pl.ds(nbase, WCH)], wbuf)
                pltpu.sync_copy(dinv_hbm.at[pl.ds(nbase, WCH)], rows_v)

                def row(r, _):
                    for i in range(F // 16):
                        sl = pl.ds(i * 16, 16)
                        v = wbuf[r, sl] * rows_v[r, sl]
                        if with_epi:
                            v = v + bias_v[sl]
                            v = jnp.where(
                                v > 0, v,
                                jnp.exp(jnp.minimum(v, 0.0)) - 1.0)
                        wbuf[r, sl] = v
                    return 0
                lax.fori_loop(0, WCH, row, 0)
                pltpu.sync_copy(wbuf, out_hbm.at[pl.ds(obase + w * WCH, WCH)])
                return 0
            lax.fori_loop(0, ROWS_W // WCH, wout, 0)
            if p + 1 < npass:
                plsc.subcore_barrier()

    return pl.kernel(
        body,
        out_type=jax.ShapeDtypeStruct((out_rows, F), jnp.float32),
        mesh=_MESH,
        scratch_types=[
            pltpu.VMEM((2 * K,), jnp.int32),
            pltpu.VMEM((K,), jnp.float32),
            pltpu.VMEM((K,), jnp.int32),
            pltpu.VMEM((K,), jnp.int32),
            pltpu.VMEM((K, F), jnp.float32),
            pltpu.VMEM((WCH, F), jnp.float32),
            pltpu.VMEM((F,), jnp.float32),
            pltpu.VMEM_SHARED((NPAD, F), jnp.float32),
            pltpu.SemaphoreType.DMA,
        ],
    )


def _matmul(h, w, b, elu, nbout, scale=None):
    nbin = h.shape[0]
    nt = 1024

    def body(h_ref, w_ref, b_ref, *rest):
        if scale is not None:
            sc_ref, o_ref = rest
        else:
            (o_ref,) = rest
        hcat = jnp.concatenate([h_ref[k] for k in range(nbin)], axis=-1)
        acc = lax.dot_general(hcat, w_ref[...], (((1,), (0,)), ((), ())),
                              preferred_element_type=jnp.float32)
        acc = acc + b_ref[...]
        if elu:
            acc = jnp.where(acc > 0, acc, jnp.exp(jnp.minimum(acc, 0.0)) - 1.0)
        for j in range(nbout):
            blk = acc[:, j * 128:(j + 1) * 128]
            if scale is not None:
                blk = blk * sc_ref[...]
            o_ref[j] = blk

    in_specs = [
        pl.BlockSpec((nbin, nt, 128), lambda i: (0, i, 0)),
        pl.BlockSpec((nbin * 128, nbout * 128), lambda i: (0, 0)),
        pl.BlockSpec((1, nbout * 128), lambda i: (0, 0)),
    ]
    args = [h, w, b]
    if scale is not None:
        in_specs.append(pl.BlockSpec((nt, 128), lambda i: (i, 0)))
        args.append(scale)
    return pl.pallas_call(
        body,
        grid=(NPAD // nt,),
        in_specs=in_specs,
        out_specs=pl.BlockSpec((nbout, nt, 128), lambda i: (0, i, 0)),
        out_shape=jax.ShapeDtypeStruct((nbout, NPAD, 128), jnp.float32),
    )(*args)


_deg_kernel = _make_deg_kernel()
_agg_split = _make_agg_kernel(split_mode=True, with_epi=False)
_agg_blocks = _make_agg_kernel(split_mode=False, with_epi=True)


def kernel(x, edge_index, weight,
           W1, b1, W2, b2, W3, b3, W4, b4, W5, b5, W6, b6, W7, b7, W8, b8,
           W9, b9):
    src = edge_index[0]
    dst = edge_index[1]
    loop = jnp.arange(N, dtype=jnp.int32)
    src2 = jnp.concatenate([src, loop])
    dst2 = jnp.concatenate([dst, loop])
    ew2 = jnp.concatenate([weight, jnp.ones((N,), jnp.float32)])
    pad = EP - E2
    srcp = jnp.pad(src2, (0, pad))
    dstp = jnp.pad(dst2, (0, pad))
    ewp = jnp.pad(ew2, (0, pad))
    meta = jnp.stack(
        [srcp.reshape(EP // K, K), dstp.reshape(EP // K, K)],
        axis=1).reshape(-1)

    deg2 = _deg_kernel(meta, ewp)
    dinv128 = _dinv_tc(deg2.reshape(2, NPAD, 128))

    zero_bias = jnp.zeros((F,), jnp.float32)
    zb1 = jnp.zeros((1, F), jnp.float32)
    xpad = jnp.pad(x, ((0, NPAD - N), (0, 0)))
    eye = jnp.eye(F, dtype=jnp.float32)

    g1 = _matmul(xpad.reshape(1, NPAD, F), eye, zb1, False, 1, scale=dinv128)
    p1 = _agg_split(g1.reshape(NPAD, F), meta, ewp, dinv128, zero_bias)
    w1s = jnp.concatenate([W1, W1], axis=0)
    h = _matmul(p1.reshape(2, NPAD, F), w1s, b1.reshape(1, -1), True, 4)

    for wl, bl in ((W2, b2), (W3, b3), (W4, b4), (W5, b5), (W6, b6),
                   (W7, b7), (W8, b8)):
        hw = _matmul(h, wl, jnp.zeros((1, 512), jnp.float32), False, 4,
                     scale=dinv128)
        h = _agg_blocks(hw.reshape(4 * NPAD, F), meta, ewp, dinv128,
                        bl).reshape(4, NPAD, F)

    w9p = jnp.pad(W9, ((0, 0), (0, F - W9.shape[1])))
    b9p = jnp.pad(b9, (0, F - b9.shape[0]))
    hw9 = _matmul(h, w9p, zb1, False, 1, scale=dinv128)
    p9 = _agg_split(hw9.reshape(NPAD, F), meta, ewp, dinv128, zero_bias)
    wf = jnp.concatenate([eye, eye], axis=0)
    out = _matmul(p9.reshape(2, NPAD, F), wf, b9p.reshape(1, -1), False, 1)
    return out[0, :N, :W9.shape[1]]
